# baseline (speedup 1.0000x reference)
import math
from functools import partial

import jax
import jax.numpy as jnp
from jax.experimental import pallas as pl
from jax.experimental.pallas import tpu as pltpu

_EPS = 1e-5

_CFG = {
    'sa1': {'ratio': 0.25, 'radius_list': [0.1, 0.2, 0.4],
            'max_sample_list': [16, 32, 128]},
    'sa2': {'ratio': 0.25, 'radius_list': [0.2, 0.4, 0.8],
            'max_sample_list': [32, 64, 128]},
    'sa3': {'ratio': 0.0078125, 'radius_list': [1000.0],
            'max_sample_list': [128]},
}


def _fps_body(npoint, xyz_ref, cent_ref, dist_ref):
    n = xyz_ref.shape[0]
    dist_ref[...] = jnp.full((n, 1), 1e10, jnp.float32)
    iota = jax.lax.broadcasted_iota(jnp.int32, (n, 1), 0)

    def step(i, far):
        c = xyz_ref[pl.ds(far, 1), :]
        d = jnp.sum((xyz_ref[...] - c) ** 2, axis=-1, keepdims=True)
        dist = jnp.minimum(dist_ref[...], d)
        dist_ref[...] = dist
        cent_ref[pl.ds(i, 1), :] = jnp.full((1, 128), far, jnp.int32)
        mx = jnp.max(dist)
        return jnp.min(jnp.where(dist >= mx, iota, n)).astype(jnp.int32)

    jax.lax.fori_loop(0, npoint, step, jnp.int32(0))


def _fps(xyz, npoint):
    b, n, _ = xyz.shape
    xpad = jnp.concatenate([xyz, jnp.zeros((b, n, 5), jnp.float32)], axis=-1)
    cent = pl.pallas_call(
        partial(_fps_body, npoint),
        grid=(b,),
        in_specs=[pl.BlockSpec((None, n, 8), lambda i: (i, 0, 0))],
        out_specs=pl.BlockSpec((None, npoint, 128), lambda i: (i, 0, 0)),
        out_shape=jax.ShapeDtypeStruct((b, npoint, 128), jnp.int32),
        scratch_shapes=[pltpu.VMEM((n, 1), jnp.float32)],
    )(xpad)
    return cent[:, :, 0]


def _mm_first_body(x_ref, w_ref, b_ref, z_ref, s_ref, ss_ref):
    z = jnp.dot(x_ref[...], w_ref[...],
                preferred_element_type=jnp.float32) + b_ref[...]
    z_ref[...] = z
    co = z.shape[1]
    s_ref[...] = jnp.broadcast_to(jnp.sum(z, axis=0, keepdims=True), (8, co))
    ss_ref[...] = jnp.broadcast_to(jnp.sum(z * z, axis=0, keepdims=True),
                                   (8, co))


def _mm_aff_body(x_ref, sc_ref, sh_ref, w_ref, b_ref, z_ref, s_ref, ss_ref):
    xa = jnp.maximum(x_ref[...] * sc_ref[...] + sh_ref[...], 0.0)
    z = jnp.dot(xa, w_ref[...],
                preferred_element_type=jnp.float32) + b_ref[...]
    z_ref[...] = z
    co = z.shape[1]
    s_ref[...] = jnp.broadcast_to(jnp.sum(z, axis=0, keepdims=True), (8, co))
    ss_ref[...] = jnp.broadcast_to(jnp.sum(z * z, axis=0, keepdims=True),
                                   (8, co))


def _aff_max_body(ns, x_ref, sc_ref, sh_ref, o_ref):
    y = jnp.maximum(x_ref[...] * sc_ref[...] + sh_ref[...], 0.0)
    mt, c = y.shape
    o_ref[...] = jnp.max(y.reshape(mt // ns, ns, c), axis=1)


def _mm_stats(x, w, b, sc=None, sh=None, mt=1024):
    m, cin = x.shape
    co = w.shape[1]
    mt = min(mt, m)
    g = m // mt
    if sc is None:
        body = _mm_first_body
        ins = (x, w, b.reshape(1, co))
        in_specs = [pl.BlockSpec((mt, cin), lambda i: (i, 0)),
                    pl.BlockSpec((cin, co), lambda i: (0, 0)),
                    pl.BlockSpec((1, co), lambda i: (0, 0))]
    else:
        body = _mm_aff_body
        ins = (x, sc, sh, w, b.reshape(1, co))
        in_specs = [pl.BlockSpec((mt, cin), lambda i: (i, 0)),
                    pl.BlockSpec((1, cin), lambda i: (0, 0)),
                    pl.BlockSpec((1, cin), lambda i: (0, 0)),
                    pl.BlockSpec((cin, co), lambda i: (0, 0)),
                    pl.BlockSpec((1, co), lambda i: (0, 0))]
    z, s, ss = pl.pallas_call(
        body,
        grid=(g,),
        in_specs=in_specs,
        out_specs=[pl.BlockSpec((mt, co), lambda i: (i, 0)),
                   pl.BlockSpec((None, 8, co), lambda i: (i, 0, 0)),
                   pl.BlockSpec((None, 8, co), lambda i: (i, 0, 0))],
        out_shape=[jax.ShapeDtypeStruct((m, co), jnp.float32),
                   jax.ShapeDtypeStruct((g, 8, co), jnp.float32),
                   jax.ShapeDtypeStruct((g, 8, co), jnp.float32)],
    )(*ins)
    return z, s[:, 0, :], ss[:, 0, :]


def _bn_affine(s, ss, m, gamma, beta):
    mean = jnp.sum(s, axis=0) / m
    var = jnp.sum(ss, axis=0) / m - mean * mean
    scale = gamma / jnp.sqrt(var + _EPS)
    shift = beta - mean * scale
    return scale.reshape(1, -1), shift.reshape(1, -1)


def _aff_max(z, sc, sh, ns, mt=1024):
    m, c = z.shape
    mt = min(mt, m)
    g = m // mt
    return pl.pallas_call(
        partial(_aff_max_body, ns),
        grid=(g,),
        in_specs=[pl.BlockSpec((mt, c), lambda i: (i, 0)),
                  pl.BlockSpec((1, c), lambda i: (0, 0)),
                  pl.BlockSpec((1, c), lambda i: (0, 0))],
        out_specs=pl.BlockSpec((mt // ns, c), lambda i: (i, 0)),
        out_shape=jax.ShapeDtypeStruct((m // ns, c), jnp.float32),
    )(z, sc, sh)


def _mlp_max(grouped_flat, layers, ns):
    m = grouped_flat.shape[0]
    x = grouped_flat
    sc = sh = None
    for layer in layers:
        x, s, ss = _mm_stats(x, layer['W'], layer['b'], sc, sh)
        sc, sh = _bn_affine(s, ss, m, layer['gamma'], layer['beta'])
    return _aff_max(x, sc, sh, ns)


def _square_distance(src, dst):
    return (jnp.sum(src ** 2, -1, keepdims=True)
            - 2.0 * jnp.einsum('bmc,bnc->bmn', src, dst)
            + jnp.sum(dst ** 2, -1)[:, None, :])


def _ball_idx(radius, nsample, xyz, new_xyz):
    b, n, _ = xyz.shape
    sqr = _square_distance(jax.lax.stop_gradient(new_xyz),
                           jax.lax.stop_gradient(xyz))
    cand = jnp.where(sqr > radius ** 2, n,
                     jnp.broadcast_to(jnp.arange(n, dtype=jnp.int32),
                                      sqr.shape))
    neg, _ = jax.lax.top_k(-cand, nsample)
    idx = -neg
    first = idx[:, :, 0:1]
    first = jnp.where(first == n, 0, first)
    return jnp.where(idx == n, jnp.broadcast_to(first, idx.shape), idx)


def _gather_pts(points, idx):
    b = points.shape[0]
    s, ns = idx.shape[1], idx.shape[2]
    flat = idx.reshape(b, -1)
    out = jnp.take_along_axis(points, flat[..., None], axis=1)
    return out.reshape(b, s, ns, points.shape[-1])


def _sa_msg(xyz, feats, cfg, scale_params):
    b, n, _ = xyz.shape
    s = int(round(cfg['ratio'] * n))
    if s == 1:
        new_xyz = xyz[:, :1, :]
    else:
        fps_idx = _fps(xyz, s)
        new_xyz = jnp.take_along_axis(xyz, fps_idx[..., None], axis=1)
    outs = []
    for radius, ns, layers in zip(cfg['radius_list'], cfg['max_sample_list'],
                                  scale_params):
        if radius >= 1000.0 and ns == n:
            g_xyz = xyz[:, None, :, :] - new_xyz[:, :, None, :]
            g_feat = feats[:, None, :, :]
        else:
            idx = _ball_idx(radius, ns, xyz, new_xyz)
            g_xyz = _gather_pts(xyz, idx) - new_xyz[:, :, None, :]
            g_feat = _gather_pts(feats, idx)
        grouped = jnp.concatenate([jnp.broadcast_to(g_feat, (b, s, ns, g_feat.shape[-1])), g_xyz], axis=-1)
        cin = grouped.shape[-1]
        out = _mlp_max(grouped.reshape(b * s * ns, cin), layers, ns)
        outs.append(out.reshape(b, s, -1))
    return new_xyz, jnp.concatenate(outs, axis=-1)


def _head_body(x_ref, w1, b1, g1, be1, w2, b2, g2, be2, w3, b3, o_ref):
    def bn_relu(h, g, be):
        mean = jnp.mean(h, axis=0, keepdims=True)
        var = jnp.mean((h - mean) ** 2, axis=0, keepdims=True)
        return jnp.maximum((h - mean) / jnp.sqrt(var + _EPS) * g[...] + be[...],
                           0.0)

    h = jnp.dot(x_ref[...], w1[...], preferred_element_type=jnp.float32) + b1[...]
    h = bn_relu(h, g1, be1)
    h = jnp.dot(h, w2[...], preferred_element_type=jnp.float32) + b2[...]
    h = bn_relu(h, g2, be2)
    o = jnp.dot(h, w3[...], preferred_element_type=jnp.float32) + b3[...]
    mx = jnp.max(o, axis=-1, keepdims=True)
    sh = o - mx
    o_ref[...] = sh - jnp.log(jnp.sum(jnp.exp(sh), axis=-1, keepdims=True))


def _head(x, params):
    b = x.shape[0]
    w1, b1 = params['fc1']['W'], params['fc1']['b']
    w2, b2 = params['fc2']['W'], params['fc2']['b']
    w3, b3 = params['fc3']['W'], params['fc3']['b']
    g1, be1 = params['bn1']['gamma'], params['bn1']['beta']
    g2, be2 = params['bn2']['gamma'], params['bn2']['beta']
    nc = w3.shape[1]
    args = (x, w1, b1.reshape(1, -1), g1.reshape(1, -1), be1.reshape(1, -1),
            w2, b2.reshape(1, -1), g2.reshape(1, -1), be2.reshape(1, -1),
            w3, b3.reshape(1, -1))
    in_specs = [pl.BlockSpec(a.shape, lambda i: (0, 0)) for a in args]
    return pl.pallas_call(
        _head_body,
        grid=(1,),
        in_specs=in_specs,
        out_specs=pl.BlockSpec((b, nc), lambda i: (0, 0)),
        out_shape=jax.ShapeDtypeStruct((b, nc), jnp.float32),
    )(*args)


def kernel(xyz, features, params):
    l1_xyz, l1_points = _sa_msg(xyz, features, _CFG['sa1'], params['sa1'])
    l2_xyz, l2_points = _sa_msg(l1_xyz, l1_points, _CFG['sa2'], params['sa2'])
    l3_xyz, l3_points = _sa_msg(l2_xyz, l2_points, _CFG['sa3'], params['sa3'])
    b = xyz.shape[0]
    x = l3_points.reshape(b, 1024)
    return _head(x, params), l3_points


# speedup vs baseline: 1.0922x; 1.0922x over previous
import math
from functools import partial

import jax
import jax.numpy as jnp
from jax.experimental import pallas as pl
from jax.experimental.pallas import tpu as pltpu

_EPS = 1e-5

_CFG = {
    'sa1': {'ratio': 0.25, 'radius_list': [0.1, 0.2, 0.4],
            'max_sample_list': [16, 32, 128]},
    'sa2': {'ratio': 0.25, 'radius_list': [0.2, 0.4, 0.8],
            'max_sample_list': [32, 64, 128]},
    'sa3': {'ratio': 0.0078125, 'radius_list': [1000.0],
            'max_sample_list': [128]},
}


def _fps_body(npoint, xyz_ref, cent_ref, dist_ref):
    _, n, b = xyz_ref.shape
    x0 = xyz_ref[0]
    x1 = xyz_ref[1]
    x2 = xyz_ref[2]
    dist_ref[...] = jnp.full((n, b), 1e10, jnp.float32)
    iota = jax.lax.broadcasted_iota(jnp.int32, (n, b), 0)

    def step(i, far):
        sel = iota == far
        c0 = jnp.sum(jnp.where(sel, x0, 0.0), axis=0, keepdims=True)
        c1 = jnp.sum(jnp.where(sel, x1, 0.0), axis=0, keepdims=True)
        c2 = jnp.sum(jnp.where(sel, x2, 0.0), axis=0, keepdims=True)
        d = (x0 - c0) ** 2 + (x1 - c1) ** 2 + (x2 - c2) ** 2
        dist = jnp.minimum(dist_ref[...], d)
        dist_ref[...] = dist
        cent_ref[pl.ds(i, 1), :] = far
        mx = jnp.max(dist, axis=0, keepdims=True)
        return jnp.min(jnp.where(dist >= mx, iota, n), axis=0, keepdims=True)

    jax.lax.fori_loop(0, npoint, step,
                      jnp.zeros((1, b), jnp.int32))


def _fps(xyz, npoint):
    b, n, _ = xyz.shape
    xt = jnp.transpose(xyz, (2, 1, 0))
    cent = pl.pallas_call(
        partial(_fps_body, npoint),
        grid=(1,),
        in_specs=[pl.BlockSpec((3, n, b), lambda i: (0, 0, 0))],
        out_specs=pl.BlockSpec((npoint, b), lambda i: (0, 0)),
        out_shape=jax.ShapeDtypeStruct((npoint, b), jnp.int32),
        scratch_shapes=[pltpu.VMEM((n, b), jnp.float32)],
    )(xt)
    return cent.T


def _mm_first_body(x_ref, w_ref, b_ref, z_ref, s_ref, ss_ref):
    z = jnp.dot(x_ref[...], w_ref[...],
                preferred_element_type=jnp.float32) + b_ref[...]
    z_ref[...] = z
    co = z.shape[1]
    s_ref[...] = jnp.broadcast_to(jnp.sum(z, axis=0, keepdims=True), (8, co))
    ss_ref[...] = jnp.broadcast_to(jnp.sum(z * z, axis=0, keepdims=True),
                                   (8, co))


def _mm_aff_body(x_ref, sc_ref, sh_ref, w_ref, b_ref, z_ref, s_ref, ss_ref):
    xa = jnp.maximum(x_ref[...] * sc_ref[...] + sh_ref[...], 0.0)
    z = jnp.dot(xa, w_ref[...],
                preferred_element_type=jnp.float32) + b_ref[...]
    z_ref[...] = z
    co = z.shape[1]
    s_ref[...] = jnp.broadcast_to(jnp.sum(z, axis=0, keepdims=True), (8, co))
    ss_ref[...] = jnp.broadcast_to(jnp.sum(z * z, axis=0, keepdims=True),
                                   (8, co))


def _aff_max_body(ns, x_ref, sc_ref, sh_ref, o_ref):
    y = jnp.maximum(x_ref[...] * sc_ref[...] + sh_ref[...], 0.0)
    mt, c = y.shape
    o_ref[...] = jnp.max(y.reshape(mt // ns, ns, c), axis=1)


def _mm_stats(x, w, b, sc=None, sh=None, mt=1024):
    m, cin = x.shape
    co = w.shape[1]
    mt = min(mt, m)
    g = m // mt
    if sc is None:
        body = _mm_first_body
        ins = (x, w, b.reshape(1, co))
        in_specs = [pl.BlockSpec((mt, cin), lambda i: (i, 0)),
                    pl.BlockSpec((cin, co), lambda i: (0, 0)),
                    pl.BlockSpec((1, co), lambda i: (0, 0))]
    else:
        body = _mm_aff_body
        ins = (x, sc, sh, w, b.reshape(1, co))
        in_specs = [pl.BlockSpec((mt, cin), lambda i: (i, 0)),
                    pl.BlockSpec((1, cin), lambda i: (0, 0)),
                    pl.BlockSpec((1, cin), lambda i: (0, 0)),
                    pl.BlockSpec((cin, co), lambda i: (0, 0)),
                    pl.BlockSpec((1, co), lambda i: (0, 0))]
    z, s, ss = pl.pallas_call(
        body,
        grid=(g,),
        in_specs=in_specs,
        out_specs=[pl.BlockSpec((mt, co), lambda i: (i, 0)),
                   pl.BlockSpec((None, 8, co), lambda i: (i, 0, 0)),
                   pl.BlockSpec((None, 8, co), lambda i: (i, 0, 0))],
        out_shape=[jax.ShapeDtypeStruct((m, co), jnp.float32),
                   jax.ShapeDtypeStruct((g, 8, co), jnp.float32),
                   jax.ShapeDtypeStruct((g, 8, co), jnp.float32)],
    )(*ins)
    return z, s[:, 0, :], ss[:, 0, :]


def _bn_affine(s, ss, m, gamma, beta):
    mean = jnp.sum(s, axis=0) / m
    var = jnp.sum(ss, axis=0) / m - mean * mean
    scale = gamma / jnp.sqrt(var + _EPS)
    shift = beta - mean * scale
    return scale.reshape(1, -1), shift.reshape(1, -1)


def _aff_max(z, sc, sh, ns, mt=1024):
    m, c = z.shape
    mt = min(mt, m)
    g = m // mt
    return pl.pallas_call(
        partial(_aff_max_body, ns),
        grid=(g,),
        in_specs=[pl.BlockSpec((mt, c), lambda i: (i, 0)),
                  pl.BlockSpec((1, c), lambda i: (0, 0)),
                  pl.BlockSpec((1, c), lambda i: (0, 0))],
        out_specs=pl.BlockSpec((mt // ns, c), lambda i: (i, 0)),
        out_shape=jax.ShapeDtypeStruct((m // ns, c), jnp.float32),
    )(z, sc, sh)


def _mlp_max(grouped_flat, layers, ns):
    m = grouped_flat.shape[0]
    x = grouped_flat
    sc = sh = None
    for layer in layers:
        x, s, ss = _mm_stats(x, layer['W'], layer['b'], sc, sh)
        sc, sh = _bn_affine(s, ss, m, layer['gamma'], layer['beta'])
    return _aff_max(x, sc, sh, ns)


def _chain_stats_body(k, x_ref, *refs):
    wb = refs[:2 * k]
    aff = refs[2 * k:2 * k + 2 * (k - 1)]
    s_ref, ss_ref = refs[-2:]
    h = x_ref[...]
    for i in range(k):
        z = jnp.dot(h, wb[2 * i][...],
                    preferred_element_type=jnp.float32) + wb[2 * i + 1][...]
        if i < k - 1:
            h = jnp.maximum(z * aff[2 * i][...] + aff[2 * i + 1][...], 0.0)
    co = z.shape[1]
    s_ref[...] = jnp.broadcast_to(jnp.sum(z, axis=0, keepdims=True), (8, co))
    ss_ref[...] = jnp.broadcast_to(jnp.sum(z * z, axis=0, keepdims=True),
                                   (8, co))


def _chain_max_body(k, ns, x_ref, *refs):
    wb = refs[:2 * k]
    aff = refs[2 * k:2 * k + 2 * k]
    o_ref = refs[-1]
    h = x_ref[...]
    for i in range(k):
        z = jnp.dot(h, wb[2 * i][...],
                    preferred_element_type=jnp.float32) + wb[2 * i + 1][...]
        h = jnp.maximum(z * aff[2 * i][...] + aff[2 * i + 1][...], 0.0)
    mt, c = h.shape
    o_ref[...] = jnp.max(h.reshape(mt // ns, ns, c), axis=1)


def _row_spec(arr, mt):
    return pl.BlockSpec((mt, arr.shape[1]), lambda i: (i, 0))


def _full_spec(arr):
    return pl.BlockSpec(arr.shape, lambda i: (0,) * arr.ndim)


def _mlp_max_fused(x, layers, ns, mt=4096):
    m = x.shape[0]
    mt = min(mt, m)
    g = m // mt
    n_l = len(layers)
    wbs = []
    for layer in layers:
        wbs += [layer['W'], layer['b'].reshape(1, -1)]
    affs = []
    for k in range(1, n_l + 1):
        co = layers[k - 1]['W'].shape[1]
        args = [x] + wbs[:2 * k] + affs
        in_specs = ([_row_spec(x, mt)]
                    + [_full_spec(a) for a in args[1:]])
        s, ss = pl.pallas_call(
            partial(_chain_stats_body, k),
            grid=(g,),
            in_specs=in_specs,
            out_specs=[pl.BlockSpec((None, 8, co), lambda i: (i, 0, 0)),
                       pl.BlockSpec((None, 8, co), lambda i: (i, 0, 0))],
            out_shape=[jax.ShapeDtypeStruct((g, 8, co), jnp.float32),
                       jax.ShapeDtypeStruct((g, 8, co), jnp.float32)],
        )(*args)
        sc, sh = _bn_affine(s[:, 0, :], ss[:, 0, :], m,
                            layers[k - 1]['gamma'], layers[k - 1]['beta'])
        affs += [sc, sh]
    c_last = layers[-1]['W'].shape[1]
    args = [x] + wbs + affs
    in_specs = [_row_spec(x, mt)] + [_full_spec(a) for a in args[1:]]
    return pl.pallas_call(
        partial(_chain_max_body, n_l, ns),
        grid=(g,),
        in_specs=in_specs,
        out_specs=pl.BlockSpec((mt // ns, c_last), lambda i: (i, 0)),
        out_shape=jax.ShapeDtypeStruct((m // ns, c_last), jnp.float32),
    )(*args)


def _square_distance(src, dst):
    return (jnp.sum(src ** 2, -1, keepdims=True)
            - 2.0 * jnp.einsum('bmc,bnc->bmn', src, dst)
            + jnp.sum(dst ** 2, -1)[:, None, :])


def _ball_idx(radius, nsample, xyz, new_xyz):
    b, n, _ = xyz.shape
    sqr = _square_distance(jax.lax.stop_gradient(new_xyz),
                           jax.lax.stop_gradient(xyz))
    cand = jnp.where(sqr > radius ** 2, n,
                     jnp.broadcast_to(jnp.arange(n, dtype=jnp.int32),
                                      sqr.shape))
    neg, _ = jax.lax.top_k(-cand, nsample)
    idx = -neg
    first = idx[:, :, 0:1]
    first = jnp.where(first == n, 0, first)
    return jnp.where(idx == n, jnp.broadcast_to(first, idx.shape), idx)


def _gather_pts(points, idx):
    b = points.shape[0]
    s, ns = idx.shape[1], idx.shape[2]
    flat = idx.reshape(b, -1)
    out = jnp.take_along_axis(points, flat[..., None], axis=1)
    return out.reshape(b, s, ns, points.shape[-1])


def _sa_msg(xyz, feats, cfg, scale_params):
    b, n, _ = xyz.shape
    s = int(round(cfg['ratio'] * n))
    if s == 1:
        new_xyz = xyz[:, :1, :]
    else:
        fps_idx = _fps(xyz, s)
        new_xyz = jnp.take_along_axis(xyz, fps_idx[..., None], axis=1)
    outs = []
    for radius, ns, layers in zip(cfg['radius_list'], cfg['max_sample_list'],
                                  scale_params):
        if radius >= 1000.0 and ns == n:
            g_xyz = xyz[:, None, :, :] - new_xyz[:, :, None, :]
            g_feat = feats[:, None, :, :]
        else:
            idx = _ball_idx(radius, ns, xyz, new_xyz)
            g_xyz = _gather_pts(xyz, idx) - new_xyz[:, :, None, :]
            g_feat = _gather_pts(feats, idx)
        grouped = jnp.concatenate([jnp.broadcast_to(g_feat, (b, s, ns, g_feat.shape[-1])), g_xyz], axis=-1)
        cin = grouped.shape[-1]
        flat = grouped.reshape(b * s * ns, cin)
        if cin <= 64:
            out = _mlp_max_fused(flat, layers, ns)
        else:
            out = _mlp_max(flat, layers, ns)
        outs.append(out.reshape(b, s, -1))
    return new_xyz, jnp.concatenate(outs, axis=-1)


def _head_body(x_ref, w1, b1, g1, be1, w2, b2, g2, be2, w3, b3, o_ref):
    def bn_relu(h, g, be):
        mean = jnp.mean(h, axis=0, keepdims=True)
        var = jnp.mean((h - mean) ** 2, axis=0, keepdims=True)
        return jnp.maximum((h - mean) / jnp.sqrt(var + _EPS) * g[...] + be[...],
                           0.0)

    h = jnp.dot(x_ref[...], w1[...], preferred_element_type=jnp.float32) + b1[...]
    h = bn_relu(h, g1, be1)
    h = jnp.dot(h, w2[...], preferred_element_type=jnp.float32) + b2[...]
    h = bn_relu(h, g2, be2)
    o = jnp.dot(h, w3[...], preferred_element_type=jnp.float32) + b3[...]
    mx = jnp.max(o, axis=-1, keepdims=True)
    sh = o - mx
    o_ref[...] = sh - jnp.log(jnp.sum(jnp.exp(sh), axis=-1, keepdims=True))


def _head(x, params):
    b = x.shape[0]
    w1, b1 = params['fc1']['W'], params['fc1']['b']
    w2, b2 = params['fc2']['W'], params['fc2']['b']
    w3, b3 = params['fc3']['W'], params['fc3']['b']
    g1, be1 = params['bn1']['gamma'], params['bn1']['beta']
    g2, be2 = params['bn2']['gamma'], params['bn2']['beta']
    nc = w3.shape[1]
    args = (x, w1, b1.reshape(1, -1), g1.reshape(1, -1), be1.reshape(1, -1),
            w2, b2.reshape(1, -1), g2.reshape(1, -1), be2.reshape(1, -1),
            w3, b3.reshape(1, -1))
    in_specs = [pl.BlockSpec(a.shape, lambda i: (0, 0)) for a in args]
    return pl.pallas_call(
        _head_body,
        grid=(1,),
        in_specs=in_specs,
        out_specs=pl.BlockSpec((b, nc), lambda i: (0, 0)),
        out_shape=jax.ShapeDtypeStruct((b, nc), jnp.float32),
    )(*args)


def _sa3_head_body(ns, x_ref, w1, b1, g1, be1, w2, b2, g2, be2, w3, b3,
                   g3, be3, hw1, hb1, hg1, hbe1, hw2, hb2, hg2, hbe2,
                   hw3, hb3, l3_ref, o_ref):
    def bn(z, g, be, relu=True):
        mean = jnp.mean(z, axis=0, keepdims=True)
        var = jnp.mean((z - mean) ** 2, axis=0, keepdims=True)
        y = (z - mean) / jnp.sqrt(var + _EPS) * g[...] + be[...]
        return jnp.maximum(y, 0.0) if relu else y

    h = x_ref[...]
    for w, bb, g, be in ((w1, b1, g1, be1), (w2, b2, g2, be2),
                         (w3, b3, g3, be3)):
        z = jnp.dot(h, w[...], preferred_element_type=jnp.float32) + bb[...]
        h = bn(z, g, be)
    m, c = h.shape
    l3 = jnp.max(h.reshape(m // ns, ns, c), axis=1)
    l3_ref[...] = l3
    h = l3
    for w, bb, g, be in ((hw1, hb1, hg1, hbe1), (hw2, hb2, hg2, hbe2)):
        z = jnp.dot(h, w[...], preferred_element_type=jnp.float32) + bb[...]
        h = bn(z, g, be)
    o = jnp.dot(h, hw3[...], preferred_element_type=jnp.float32) + hb3[...]
    mx = jnp.max(o, axis=-1, keepdims=True)
    sh = o - mx
    o_ref[...] = sh - jnp.log(jnp.sum(jnp.exp(sh), axis=-1, keepdims=True))


def _sa3_head(xyz, feats, layers, params):
    b, n, _ = xyz.shape
    new_xyz = xyz[:, :1, :]
    grouped = jnp.concatenate([feats, xyz - new_xyz], axis=-1)
    x = grouped.reshape(b * n, -1)
    args = [x]
    for layer in layers:
        args += [layer['W'], layer['b'].reshape(1, -1),
                 layer['gamma'].reshape(1, -1), layer['beta'].reshape(1, -1)]
    for fc, bn_name in (('fc1', 'bn1'), ('fc2', 'bn2')):
        args += [params[fc]['W'], params[fc]['b'].reshape(1, -1),
                 params[bn_name]['gamma'].reshape(1, -1),
                 params[bn_name]['beta'].reshape(1, -1)]
    args += [params['fc3']['W'], params['fc3']['b'].reshape(1, -1)]
    c_last = layers[-1]['W'].shape[1]
    nc = params['fc3']['W'].shape[1]
    l3, logits = pl.pallas_call(
        partial(_sa3_head_body, n),
        grid=(1,),
        in_specs=[_full_spec(a) for a in args],
        out_specs=[pl.BlockSpec((b, c_last), lambda i: (0, 0)),
                   pl.BlockSpec((b, nc), lambda i: (0, 0))],
        out_shape=[jax.ShapeDtypeStruct((b, c_last), jnp.float32),
                   jax.ShapeDtypeStruct((b, nc), jnp.float32)],
    )(*args)
    return logits, l3.reshape(b, 1, c_last)


def kernel(xyz, features, params):
    l1_xyz, l1_points = _sa_msg(xyz, features, _CFG['sa1'], params['sa1'])
    l2_xyz, l2_points = _sa_msg(l1_xyz, l1_points, _CFG['sa2'], params['sa2'])
    logits, l3_points = _sa3_head(l2_xyz, l2_points, params['sa3'][0], params)
    return logits, l3_points
